# baseline (speedup 1.0000x reference)
import jax
import jax.numpy as jnp
from jax.experimental import pallas as pl


def _gat_conv(x, edge_index, W, a_src, a_dst, b, heads, C):
    n = x.shape[0]
    loop = jnp.arange(n, dtype=edge_index.dtype)
    src = jnp.concatenate([edge_index[0], loop])
    dst = jnp.concatenate([edge_index[1], loop])
    h = (x @ W).reshape(n, heads, C)
    alpha_src = (h * a_src).sum(-1)
    alpha_dst = (h * a_dst).sum(-1)
    alpha = alpha_src[src] + alpha_dst[dst]
    alpha = jax.nn.leaky_relu(alpha, negative_slope=0.2)
    amax = jax.ops.segment_max(alpha, dst, num_segments=n)
    alpha = jnp.exp(alpha - amax[dst])
    denom = jax.ops.segment_sum(alpha, dst, num_segments=n)
    alpha = alpha / (denom[dst] + 1e-16)
    msg = h[src] * alpha[:, :, None]
    out = jax.ops.segment_sum(msg, dst, num_segments=n)
    out = out.reshape(n, heads * C)
    return out + b


def _mlp_kernel(h_ref, wl1_ref, bl1_ref, wl2_ref, bl2_ref, o_ref):
    h = h_ref[...]
    z = jnp.maximum(h @ wl1_ref[...] + bl1_ref[...], 0.0)
    o_ref[...] = z @ wl2_ref[...] + bl2_ref[...]


def kernel(x, edge_index, W1, a_src1, a_dst1, b1, W2, a_src2, a_dst2, b2, Wl1, bl1, Wl2, bl2):
    h = _gat_conv(x, edge_index, W1, a_src1, a_dst1, b1, 8, 64)
    h = jax.nn.elu(h)
    h = _gat_conv(h, edge_index, W2, a_src2, a_dst2, b2, 1, 32)
    out = pl.pallas_call(
        _mlp_kernel,
        out_shape=jax.ShapeDtypeStruct((h.shape[0], 1), jnp.float32),
        grid=(50,),
        in_specs=[
            pl.BlockSpec((1000, 32), lambda i: (i, 0)),
            pl.BlockSpec((32, 16), lambda i: (0, 0)),
            pl.BlockSpec((16,), lambda i: (0,)),
            pl.BlockSpec((16, 1), lambda i: (0, 0)),
            pl.BlockSpec((1,), lambda i: (0,)),
        ],
        out_specs=pl.BlockSpec((1000, 1), lambda i: (i, 0)),
    )(h, Wl1, bl1, Wl2, bl2)
    return out


# speedup vs baseline: 5.4349x; 5.4349x over previous
import functools

import jax
import jax.numpy as jnp
from jax import lax
from jax.experimental import pallas as pl
from jax.experimental.pallas import tpu as pltpu
from jax.experimental.pallas import tpu_sc as plsc

BN = 256
NPAD = 53248
EPAD = 851968
W_WIN = 128
W_SLAB = 16
NB = 13
CELL_W = 37
PK_SENT = 1 << 28
PROWS = NB * 16 * CELL_W


def _prep1_body(x_ref, w_ref, asv_ref, adv_ref, h_ref, asT_ref, adT_ref):
    hb = jnp.dot(x_ref[...], w_ref[...], preferred_element_type=jnp.float32)
    h_ref[...] = hb
    h3 = hb.reshape(BN, 8, 64)
    asb = (h3 * asv_ref[...][None]).sum(-1)
    adb = (h3 * adv_ref[...][None]).sum(-1)
    asT_ref[...] = asb.T
    adT_ref[...] = adb.T


def _mid_body(num_ref, den_ref, b1_ref, w2_ref, as2v_ref, ad2v_ref,
              h2_ref, as2_ref, ad2_ref):
    den = den_ref[...].sum(-1)
    inv = 1.0 / (den + 1e-16)
    pieces = []
    for s in range(32):
        pieces.append(num_ref[s] * inv[s // 4][:, None])
    h1g = jnp.concatenate(pieces, axis=-1) + b1_ref[...][None]
    h1g = jnp.where(h1g > 0, h1g, jnp.exp(jnp.minimum(h1g, 0.0)) - 1.0)
    h2 = jnp.dot(h1g, w2_ref[...], preferred_element_type=jnp.float32)
    h2_ref[...] = h2
    as2_ref[...] = (h2 * as2v_ref[...]).sum(-1)
    ad2_ref[...] = (h2 * ad2v_ref[...]).sum(-1)


def _fin_body(num_ref, den_ref, b2_ref, wl1_ref, bl1_ref, wl2_ref, bl2_ref,
              o_ref):
    num = jnp.concatenate([num_ref[0], num_ref[1]], axis=-1)
    den = den_ref[0].sum(-1)
    h = num * (1.0 / (den + 1e-16))[:, None] + b2_ref[...][None]
    z = jnp.maximum(
        jnp.dot(h, wl1_ref[...], preferred_element_type=jnp.float32)
        + bl1_ref[...][None], 0.0)
    o_ref[...] = (jnp.dot(z, wl2_ref[...], preferred_element_type=jnp.float32)
                  + bl2_ref[...][None])


def _part_body(src2, dst2, ep, m1, sbuf, dbuf, cellb, sta, cnt_s, sem):
    t = lax.axis_index("s")
    sent = jnp.full((16,), PK_SENT, jnp.int32)

    def _prefill():
        @pl.loop(0, CELL_W)
        def _(r):
            for b in range(NB):
                for cc in range(0, W_WIN, 16):
                    sta[b, r, pl.ds(cc, 16)] = sent

    _prefill()
    for b in range(NB):
        cnt_s[b] = 0

    @pl.loop(0, 416)
    def _sb(sb):
        rowbase = t * 416 + sb
        pltpu.sync_copy(src2.at[pl.ds(rowbase, 1)], sbuf)
        pltpu.sync_copy(dst2.at[pl.ds(rowbase, 1)], dbuf)
        for cc in range(0, W_WIN, 16):
            sv = sbuf[0, pl.ds(cc, 16)]
            dv = dbuf[0, pl.ds(cc, 16)]
            pk = sv | ((dv & 4095) << 16)
            bv = dv >> 12
            for b in range(NB):
                m = bv == b
                pos = plsc.cumsum(jnp.where(m, 1, 0))
                cnt = cnt_s[b]
                idx = cnt + pos - 1
                plsc.store_scatter(sta, [jnp.full((16,), b, jnp.int32),
                                         idx >> 7, idx & 127], pk, mask=m)
                cnt_s[b] = cnt + pos[15]

    for b in range(NB):
        pltpu.sync_copy(sta.at[b], m1.at[pl.ds((b * 16 + t) * CELL_W, CELL_W)])

    plsc.subcore_barrier()

    _prefill()
    for b in range(NB):
        cnt_s[b] = 0

    for b in range(NB):
        @pl.loop(0, 16)
        def _cell(u):
            pltpu.sync_copy(m1.at[pl.ds((b * 16 + u) * CELL_W, CELL_W)],
                            cellb)

            @pl.loop(0, CELL_W)
            def _row(r):
                for cc in range(0, W_WIN, 16):
                    pk = cellb[r, pl.ds(cc, 16)]
                    m = ((pk >> 28) == 0) & (((pk >> 24) & 15) == t)
                    pos = plsc.cumsum(jnp.where(m, 1, 0))
                    cnt = cnt_s[b]
                    idx = cnt + pos - 1
                    plsc.store_scatter(sta, [jnp.full((16,), b, jnp.int32),
                                             idx >> 7, idx & 127], pk,
                                       mask=m)
                    cnt_s[b] = cnt + pos[15]

    for b in range(NB):
        pltpu.sync_copy(sta.at[b], ep.at[pl.ds((b * 16 + t) * CELL_W, CELL_W)])


def _make_gat_body(mult, nslab):
    def body(ep, asT3, adT3, hv, z2d,
             numer, den,
             pbuf, gbuf, r8buf, rowbuf, wbuf, as_t, ad_t, acc, densw, sem):
        c = lax.axis_index("c")
        t = lax.axis_index("s")
        iota = lax.iota(jnp.int32, 16)

        def _bucket(bi, carry):
            b = bi * 2 + c
            base = b * 4096

            @pl.loop(0, nslab)
            def _slab(sl):
                head = sl // 4
                pltpu.sync_copy(asT3.at[head], as_t)
                pltpu.sync_copy(adT3.at[head], ad_t)
                pltpu.sync_copy(z2d, acc)

                @pl.when(sl % 4 == 0)
                def _():
                    pltpu.sync_copy(z2d, densw)

                pltpu.sync_copy(ep.at[pl.ds((b * 16 + t) * CELL_W, CELL_W)],
                                pbuf)

                @pl.loop(0, CELL_W)
                def _gidx(r):
                    for cc in range(0, W_WIN, 16):
                        pk = pbuf[r, pl.ds(cc, 16)]
                        gbuf[r, pl.ds(cc, 16)] = (pk & 65535) * mult + sl
                        r8buf[r, pl.ds(cc, 16)] = (pk >> 16) & 255

                @pl.loop(0, CELL_W)
                def _win(wi):
                    fut = pltpu.async_copy(hv.at[gbuf.at[wi]], rowbuf, sem)
                    for j in range(8):
                        pk = pbuf[wi, pl.ds(j * 16, 16)]
                        sv = pk & 65535
                        dv = base + ((pk >> 16) & 4095)
                        a = plsc.load_gather(as_t, [sv >> 7, sv & 127])
                        bq = plsc.load_gather(ad_t, [dv >> 7, dv & 127])
                        tt = a + bq
                        tt = jnp.where(tt >= 0, tt, tt * jnp.float32(0.2))
                        wm = jnp.where((pk >> 28) == 0, jnp.float32(1.0),
                                       jnp.float32(0.0))
                        wbuf[pl.ds(j * 16, 16)] = jnp.exp(tt) * wm
                    fut.wait()
                    for g in range(8):
                        wv = wbuf[pl.ds(g * 16, 16)]
                        rv = r8buf[wi, pl.ds(g * 16, 16)]

                        @pl.when(sl % 4 == 0)
                        def _():
                            plsc.addupdate_scatter(densw, [rv, iota], wv)

                        for j in range(16):
                            r = g * 16 + j
                            wsc = wv[j]
                            rows = jnp.full((16,), rv[j], jnp.int32)
                            v = rowbuf[r, pl.ds(0, 16)] * wsc
                            plsc.addupdate_scatter(acc, [rows, iota], v)

                pltpu.sync_copy(acc,
                                numer.at[sl, pl.ds(base + t * 256, 256)])

                @pl.when(sl % 4 == 0)
                def _():
                    pltpu.sync_copy(densw,
                                    den.at[head, pl.ds(base + t * 256, 256)])

            return carry

        lax.fori_loop(0, 7 - c, _bucket, 0)

    return body


_SC_MESH = plsc.VectorSubcoreMesh(core_axis_name="c", subcore_axis_name="s")
_SC_MESH1 = plsc.VectorSubcoreMesh(core_axis_name="c", subcore_axis_name="s",
                                   num_cores=1)

_SC_PARAMS = pltpu.CompilerParams(needs_layout_passes=False,
                                  use_tc_tiling_on_sc=False)

_partition = functools.partial(
    pl.kernel, _part_body, mesh=_SC_MESH1, compiler_params=_SC_PARAMS,
    name="gat_partition",
    out_type=(jax.ShapeDtypeStruct((PROWS, W_WIN), jnp.int32),
              jax.ShapeDtypeStruct((PROWS, W_WIN), jnp.int32)),
    scratch_types=[
        pltpu.VMEM((1, W_WIN), jnp.int32),
        pltpu.VMEM((1, W_WIN), jnp.int32),
        pltpu.VMEM((CELL_W, W_WIN), jnp.int32),
        pltpu.VMEM((NB, CELL_W, W_WIN), jnp.int32),
        pltpu.SMEM((16,), jnp.int32),
        pltpu.SemaphoreType.DMA,
    ])


def _gat_scratch():
    return [
        pltpu.VMEM((CELL_W, W_WIN), jnp.int32),
        pltpu.VMEM((CELL_W, W_WIN), jnp.int32),
        pltpu.VMEM((CELL_W, W_WIN), jnp.int32),
        pltpu.VMEM((W_WIN, W_SLAB), jnp.float32),
        pltpu.VMEM((W_WIN,), jnp.float32),
        pltpu.VMEM((NPAD // 128, 128), jnp.float32),
        pltpu.VMEM((NPAD // 128, 128), jnp.float32),
        pltpu.VMEM((256, W_SLAB), jnp.float32),
        pltpu.VMEM((256, W_SLAB), jnp.float32),
        pltpu.SemaphoreType.DMA,
    ]


_gat1 = functools.partial(
    pl.kernel, _make_gat_body(32, 32), mesh=_SC_MESH,
    compiler_params=_SC_PARAMS, name="gat_edges_l1",
    out_type=(jax.ShapeDtypeStruct((32, NPAD, W_SLAB), jnp.float32),
              jax.ShapeDtypeStruct((8, NPAD, W_SLAB), jnp.float32)),
    scratch_types=_gat_scratch())

_gat2 = functools.partial(
    pl.kernel, _make_gat_body(2, 2), mesh=_SC_MESH,
    compiler_params=_SC_PARAMS, name="gat_edges_l2",
    out_type=(jax.ShapeDtypeStruct((2, NPAD, W_SLAB), jnp.float32),
              jax.ShapeDtypeStruct((1, NPAD, W_SLAB), jnp.float32)),
    scratch_types=_gat_scratch())


def kernel(x, edge_index, W1, a_src1, a_dst1, b1, W2, a_src2, a_dst2, b2,
           Wl1, bl1, Wl2, bl2):
    n = x.shape[0]
    e = edge_index.shape[1]
    grid = NPAD // BN

    loop = jnp.arange(n, dtype=jnp.int32)
    npad_e = EPAD - (e + n)
    pad_ar = jnp.arange(npad_e, dtype=jnp.int32) % 128
    src_full = jnp.concatenate([edge_index[0].astype(jnp.int32), loop, pad_ar])
    dst_full = jnp.concatenate([edge_index[1].astype(jnp.int32), loop,
                                n + pad_ar])
    src2 = src_full.reshape(EPAD // W_WIN, W_WIN)
    dst2 = dst_full.reshape(EPAD // W_WIN, W_WIN)

    xp = jnp.pad(x, ((0, NPAD - n), (0, 0)))

    ep, _m1 = _partition()(src2, dst2)

    h1, asT, adT = pl.pallas_call(
        _prep1_body,
        out_shape=(jax.ShapeDtypeStruct((NPAD, 512), jnp.float32),
                   jax.ShapeDtypeStruct((8, NPAD), jnp.float32),
                   jax.ShapeDtypeStruct((8, NPAD), jnp.float32)),
        grid=(grid,),
        in_specs=[
            pl.BlockSpec((BN, 75), lambda i: (i, 0)),
            pl.BlockSpec((75, 512), lambda i: (0, 0)),
            pl.BlockSpec((8, 64), lambda i: (0, 0)),
            pl.BlockSpec((8, 64), lambda i: (0, 0)),
        ],
        out_specs=(
            pl.BlockSpec((BN, 512), lambda i: (i, 0)),
            pl.BlockSpec((8, BN), lambda i: (0, i)),
            pl.BlockSpec((8, BN), lambda i: (0, i)),
        ),
    )(xp, W1, a_src1, a_dst1)

    h1v = h1.reshape(NPAD * 32, W_SLAB)
    z2d = jnp.zeros((256, W_SLAB), jnp.float32)
    asT3 = asT.reshape(8, NPAD // 128, 128)
    adT3 = adT.reshape(8, NPAD // 128, 128)
    numer1, den1 = _gat1()(ep, asT3, adT3, h1v, z2d)

    h2, as2, ad2 = pl.pallas_call(
        _mid_body,
        out_shape=(jax.ShapeDtypeStruct((NPAD, 32), jnp.float32),
                   jax.ShapeDtypeStruct((NPAD,), jnp.float32),
                   jax.ShapeDtypeStruct((NPAD,), jnp.float32)),
        grid=(grid,),
        in_specs=[
            pl.BlockSpec((32, BN, W_SLAB), lambda i: (0, i, 0)),
            pl.BlockSpec((8, BN, W_SLAB), lambda i: (0, i, 0)),
            pl.BlockSpec((512,), lambda i: (0,)),
            pl.BlockSpec((512, 32), lambda i: (0, 0)),
            pl.BlockSpec((1, 32), lambda i: (0, 0)),
            pl.BlockSpec((1, 32), lambda i: (0, 0)),
        ],
        out_specs=(
            pl.BlockSpec((BN, 32), lambda i: (i, 0)),
            pl.BlockSpec((BN,), lambda i: (i,)),
            pl.BlockSpec((BN,), lambda i: (i,)),
        ),
    )(numer1, den1, b1, W2, a_src2, a_dst2)

    h2v = h2.reshape(NPAD * 2, W_SLAB)
    asT3_2 = as2.reshape(1, NPAD // 128, 128)
    adT3_2 = ad2.reshape(1, NPAD // 128, 128)
    numer2, den2 = _gat2()(ep, asT3_2, adT3_2, h2v, z2d)

    out = pl.pallas_call(
        _fin_body,
        out_shape=jax.ShapeDtypeStruct((NPAD, 1), jnp.float32),
        grid=(grid,),
        in_specs=[
            pl.BlockSpec((2, BN, W_SLAB), lambda i: (0, i, 0)),
            pl.BlockSpec((1, BN, W_SLAB), lambda i: (0, i, 0)),
            pl.BlockSpec((32,), lambda i: (0,)),
            pl.BlockSpec((32, 16), lambda i: (0, 0)),
            pl.BlockSpec((16,), lambda i: (0,)),
            pl.BlockSpec((16, 1), lambda i: (0, 0)),
            pl.BlockSpec((1,), lambda i: (0,)),
        ],
        out_specs=pl.BlockSpec((BN, 1), lambda i: (i, 0)),
    )(numer2, den2, b2, Wl1, bl1, Wl2, bl2)

    return out[:n]


# speedup vs baseline: 5.9865x; 1.1015x over previous
import functools

import jax
import jax.numpy as jnp
from jax import lax
from jax.experimental import pallas as pl
from jax.experimental.pallas import tpu as pltpu
from jax.experimental.pallas import tpu_sc as plsc

BN = 256
NPAD = 53248
EPAD = 851968
W_WIN = 128
W_SLAB = 16
NB = 13
CELL_W = 37
PK_SENT = 1 << 28
PROWS = NB * 16 * CELL_W


def _prep1_body(x_ref, w_ref, asv_ref, adv_ref, h_ref, asT_ref, adT_ref):
    hb = jnp.dot(x_ref[...], w_ref[...], preferred_element_type=jnp.float32)
    h_ref[...] = hb
    h3 = hb.reshape(BN, 8, 64)
    asb = (h3 * asv_ref[...][None]).sum(-1)
    adb = (h3 * adv_ref[...][None]).sum(-1)
    asT_ref[...] = asb.T
    adT_ref[...] = adb.T


def _mid_body(num_ref, den_ref, b1_ref, w2_ref, as2v_ref, ad2v_ref,
              h2_ref, as2_ref, ad2_ref):
    den = den_ref[...].sum(-1)
    inv = 1.0 / (den + 1e-16)
    pieces = []
    for s in range(32):
        pieces.append(num_ref[s] * inv[s // 4][:, None])
    h1g = jnp.concatenate(pieces, axis=-1) + b1_ref[...][None]
    h1g = jnp.where(h1g > 0, h1g, jnp.exp(jnp.minimum(h1g, 0.0)) - 1.0)
    h2 = jnp.dot(h1g, w2_ref[...], preferred_element_type=jnp.float32)
    h2_ref[...] = h2
    as2_ref[...] = (h2 * as2v_ref[...]).sum(-1)
    ad2_ref[...] = (h2 * ad2v_ref[...]).sum(-1)


def _fin_body(num_ref, den_ref, b2_ref, wl1_ref, bl1_ref, wl2_ref, bl2_ref,
              o_ref):
    num = jnp.concatenate([num_ref[0], num_ref[1]], axis=-1)
    den = den_ref[0].sum(-1)
    h = num * (1.0 / (den + 1e-16))[:, None] + b2_ref[...][None]
    z = jnp.maximum(
        jnp.dot(h, wl1_ref[...], preferred_element_type=jnp.float32)
        + bl1_ref[...][None], 0.0)
    o_ref[...] = (jnp.dot(z, wl2_ref[...], preferred_element_type=jnp.float32)
                  + bl2_ref[...][None])


def _part_body(src2, dst2, ep, m1, sbuf, dbuf, cellb, sta, cnt_s, sem):
    t = lax.axis_index("s")
    sent = jnp.full((16,), PK_SENT, jnp.int32)

    def _prefill():
        @pl.loop(0, CELL_W)
        def _(r):
            for b in range(NB):
                for cc in range(0, W_WIN, 16):
                    sta[b, r, pl.ds(cc, 16)] = sent

    _prefill()
    for b in range(NB):
        cnt_s[b] = 0

    @pl.loop(0, 52)
    def _sb(sb):
        rowbase = t * 416 + sb * 8
        pltpu.sync_copy(src2.at[pl.ds(rowbase, 8)], sbuf)
        pltpu.sync_copy(dst2.at[pl.ds(rowbase, 8)], dbuf)

        @pl.loop(0, 8)
        def _rw(r):
          for cc in range(0, W_WIN, 16):
            sv = sbuf[r, pl.ds(cc, 16)]
            dv = dbuf[r, pl.ds(cc, 16)]
            pk = sv | ((dv & 4095) << 16)
            bv = dv >> 12
            for b in range(NB):
                m = bv == b
                pos = plsc.cumsum(jnp.where(m, 1, 0))
                cnt = cnt_s[b]
                idx = cnt + pos - 1
                plsc.store_scatter(sta, [jnp.full((16,), b, jnp.int32),
                                         idx >> 7, idx & 127], pk, mask=m)
                cnt_s[b] = cnt + pos[15]

    for b in range(NB):
        pltpu.sync_copy(sta.at[b], m1.at[pl.ds((b * 16 + t) * CELL_W, CELL_W)])

    plsc.subcore_barrier()

    _prefill()
    for b in range(NB):
        cnt_s[b] = 0

    for b in range(NB):
        @pl.loop(0, 16)
        def _cell(u):
            pltpu.sync_copy(m1.at[pl.ds((b * 16 + u) * CELL_W, CELL_W)],
                            cellb)

            @pl.loop(0, CELL_W)
            def _row(r):
                for cc in range(0, W_WIN, 16):
                    pk = cellb[r, pl.ds(cc, 16)]
                    m = ((pk >> 28) == 0) & (((pk >> 24) & 15) == t)
                    pos = plsc.cumsum(jnp.where(m, 1, 0))
                    cnt = cnt_s[b]
                    idx = cnt + pos - 1
                    plsc.store_scatter(sta, [jnp.full((16,), b, jnp.int32),
                                             idx >> 7, idx & 127], pk,
                                       mask=m)
                    cnt_s[b] = cnt + pos[15]

    for b in range(NB):
        pltpu.sync_copy(sta.at[b], ep.at[pl.ds((b * 16 + t) * CELL_W, CELL_W)])


def _make_gat_body(mult, nslab):
    def body(ep, asT3, adT3, hv, z2d,
             numer, den,
             pbuf, gbuf, r8buf, rowbuf, wbuf, as_t, ad_t, acc, densw, sem):
        c = lax.axis_index("c")
        t = lax.axis_index("s")
        iota = lax.iota(jnp.int32, 16)

        @pl.loop(0, nslab)
        def _slab(sl):
            head = sl // 4
            pltpu.sync_copy(asT3.at[head], as_t)
            pltpu.sync_copy(adT3.at[head], ad_t)

            def _bucket(bi, carry):
                b = bi * 2 + c
                base = b * 4096
                pltpu.sync_copy(z2d, acc)

                @pl.when(sl % 4 == 0)
                def _():
                    pltpu.sync_copy(z2d, densw)

                pltpu.sync_copy(ep.at[pl.ds((b * 16 + t) * CELL_W, CELL_W)],
                                pbuf)

                @pl.loop(0, CELL_W)
                def _gidx(r):
                    for cc in range(0, W_WIN, 16):
                        pk = pbuf[r, pl.ds(cc, 16)]
                        gbuf[r, pl.ds(cc, 16)] = (pk & 65535) * mult + sl
                        r8buf[r, pl.ds(cc, 16)] = (pk >> 16) & 255

                @pl.loop(0, CELL_W)
                def _win(wi):
                    fut = pltpu.async_copy(hv.at[gbuf.at[wi]], rowbuf, sem)
                    for j in range(8):
                        pk = pbuf[wi, pl.ds(j * 16, 16)]
                        sv = pk & 65535
                        dv = base + ((pk >> 16) & 4095)
                        a = plsc.load_gather(as_t, [sv >> 7, sv & 127])
                        bq = plsc.load_gather(ad_t, [dv >> 7, dv & 127])
                        tt = a + bq
                        tt = jnp.where(tt >= 0, tt, tt * jnp.float32(0.2))
                        wm = jnp.where((pk >> 28) == 0, jnp.float32(1.0),
                                       jnp.float32(0.0))
                        wbuf[pl.ds(j * 16, 16)] = jnp.exp(tt) * wm
                    fut.wait()
                    for g in range(8):
                        wv = wbuf[pl.ds(g * 16, 16)]
                        rv = r8buf[wi, pl.ds(g * 16, 16)]

                        @pl.when(sl % 4 == 0)
                        def _():
                            plsc.addupdate_scatter(densw, [rv, iota], wv)

                        for j in range(16):
                            r = g * 16 + j
                            wsc = wv[j]
                            rows = jnp.full((16,), rv[j], jnp.int32)
                            v = rowbuf[r, pl.ds(0, 16)] * wsc
                            plsc.addupdate_scatter(acc, [rows, iota], v)

                pltpu.sync_copy(acc,
                                numer.at[sl, pl.ds(base + t * 256, 256)])

                @pl.when(sl % 4 == 0)
                def _():
                    pltpu.sync_copy(densw,
                                    den.at[head, pl.ds(base + t * 256, 256)])

                return carry

            lax.fori_loop(0, 7 - c, _bucket, 0)

    return body


_SC_MESH = plsc.VectorSubcoreMesh(core_axis_name="c", subcore_axis_name="s")
_SC_MESH1 = plsc.VectorSubcoreMesh(core_axis_name="c", subcore_axis_name="s",
                                   num_cores=1)

_SC_PARAMS = pltpu.CompilerParams(needs_layout_passes=False,
                                  use_tc_tiling_on_sc=False)

_partition = functools.partial(
    pl.kernel, _part_body, mesh=_SC_MESH1, compiler_params=_SC_PARAMS,
    name="gat_partition",
    out_type=(jax.ShapeDtypeStruct((PROWS, W_WIN), jnp.int32),
              jax.ShapeDtypeStruct((PROWS, W_WIN), jnp.int32)),
    scratch_types=[
        pltpu.VMEM((8, W_WIN), jnp.int32),
        pltpu.VMEM((8, W_WIN), jnp.int32),
        pltpu.VMEM((CELL_W, W_WIN), jnp.int32),
        pltpu.VMEM((NB, CELL_W, W_WIN), jnp.int32),
        pltpu.SMEM((16,), jnp.int32),
        pltpu.SemaphoreType.DMA,
    ])


def _gat_scratch():
    return [
        pltpu.VMEM((CELL_W, W_WIN), jnp.int32),
        pltpu.VMEM((CELL_W, W_WIN), jnp.int32),
        pltpu.VMEM((CELL_W, W_WIN), jnp.int32),
        pltpu.VMEM((W_WIN, W_SLAB), jnp.float32),
        pltpu.VMEM((W_WIN,), jnp.float32),
        pltpu.VMEM((NPAD // 128, 128), jnp.float32),
        pltpu.VMEM((NPAD // 128, 128), jnp.float32),
        pltpu.VMEM((256, W_SLAB), jnp.float32),
        pltpu.VMEM((256, W_SLAB), jnp.float32),
        pltpu.SemaphoreType.DMA,
    ]


_gat1 = functools.partial(
    pl.kernel, _make_gat_body(32, 32), mesh=_SC_MESH,
    compiler_params=_SC_PARAMS, name="gat_edges_l1",
    out_type=(jax.ShapeDtypeStruct((32, NPAD, W_SLAB), jnp.float32),
              jax.ShapeDtypeStruct((8, NPAD, W_SLAB), jnp.float32)),
    scratch_types=_gat_scratch())

_gat2 = functools.partial(
    pl.kernel, _make_gat_body(2, 2), mesh=_SC_MESH,
    compiler_params=_SC_PARAMS, name="gat_edges_l2",
    out_type=(jax.ShapeDtypeStruct((2, NPAD, W_SLAB), jnp.float32),
              jax.ShapeDtypeStruct((1, NPAD, W_SLAB), jnp.float32)),
    scratch_types=_gat_scratch())


def kernel(x, edge_index, W1, a_src1, a_dst1, b1, W2, a_src2, a_dst2, b2,
           Wl1, bl1, Wl2, bl2):
    n = x.shape[0]
    e = edge_index.shape[1]
    grid = NPAD // BN

    loop = jnp.arange(n, dtype=jnp.int32)
    npad_e = EPAD - (e + n)
    pad_ar = jnp.arange(npad_e, dtype=jnp.int32) % 128
    src_full = jnp.concatenate([edge_index[0].astype(jnp.int32), loop, pad_ar])
    dst_full = jnp.concatenate([edge_index[1].astype(jnp.int32), loop,
                                n + pad_ar])
    src2 = src_full.reshape(EPAD // W_WIN, W_WIN)
    dst2 = dst_full.reshape(EPAD // W_WIN, W_WIN)

    xp = jnp.pad(x, ((0, NPAD - n), (0, 0)))

    ep, _m1 = _partition()(src2, dst2)

    h1, asT, adT = pl.pallas_call(
        _prep1_body,
        out_shape=(jax.ShapeDtypeStruct((NPAD, 512), jnp.float32),
                   jax.ShapeDtypeStruct((8, NPAD), jnp.float32),
                   jax.ShapeDtypeStruct((8, NPAD), jnp.float32)),
        grid=(grid,),
        in_specs=[
            pl.BlockSpec((BN, 75), lambda i: (i, 0)),
            pl.BlockSpec((75, 512), lambda i: (0, 0)),
            pl.BlockSpec((8, 64), lambda i: (0, 0)),
            pl.BlockSpec((8, 64), lambda i: (0, 0)),
        ],
        out_specs=(
            pl.BlockSpec((BN, 512), lambda i: (i, 0)),
            pl.BlockSpec((8, BN), lambda i: (0, i)),
            pl.BlockSpec((8, BN), lambda i: (0, i)),
        ),
    )(xp, W1, a_src1, a_dst1)

    h1v = h1.reshape(NPAD * 32, W_SLAB)
    z2d = jnp.zeros((256, W_SLAB), jnp.float32)
    asT3 = asT.reshape(8, NPAD // 128, 128)
    adT3 = adT.reshape(8, NPAD // 128, 128)
    numer1, den1 = _gat1()(ep, asT3, adT3, h1v, z2d)

    h2, as2, ad2 = pl.pallas_call(
        _mid_body,
        out_shape=(jax.ShapeDtypeStruct((NPAD, 32), jnp.float32),
                   jax.ShapeDtypeStruct((NPAD,), jnp.float32),
                   jax.ShapeDtypeStruct((NPAD,), jnp.float32)),
        grid=(grid,),
        in_specs=[
            pl.BlockSpec((32, BN, W_SLAB), lambda i: (0, i, 0)),
            pl.BlockSpec((8, BN, W_SLAB), lambda i: (0, i, 0)),
            pl.BlockSpec((512,), lambda i: (0,)),
            pl.BlockSpec((512, 32), lambda i: (0, 0)),
            pl.BlockSpec((1, 32), lambda i: (0, 0)),
            pl.BlockSpec((1, 32), lambda i: (0, 0)),
        ],
        out_specs=(
            pl.BlockSpec((BN, 32), lambda i: (i, 0)),
            pl.BlockSpec((BN,), lambda i: (i,)),
            pl.BlockSpec((BN,), lambda i: (i,)),
        ),
    )(numer1, den1, b1, W2, a_src2, a_dst2)

    h2v = h2.reshape(NPAD * 2, W_SLAB)
    asT3_2 = as2.reshape(1, NPAD // 128, 128)
    adT3_2 = ad2.reshape(1, NPAD // 128, 128)
    numer2, den2 = _gat2()(ep, asT3_2, adT3_2, h2v, z2d)

    out = pl.pallas_call(
        _fin_body,
        out_shape=jax.ShapeDtypeStruct((NPAD, 1), jnp.float32),
        grid=(grid,),
        in_specs=[
            pl.BlockSpec((2, BN, W_SLAB), lambda i: (0, i, 0)),
            pl.BlockSpec((1, BN, W_SLAB), lambda i: (0, i, 0)),
            pl.BlockSpec((32,), lambda i: (0,)),
            pl.BlockSpec((32, 16), lambda i: (0, 0)),
            pl.BlockSpec((16,), lambda i: (0,)),
            pl.BlockSpec((16, 1), lambda i: (0, 0)),
            pl.BlockSpec((1,), lambda i: (0,)),
        ],
        out_specs=pl.BlockSpec((BN, 1), lambda i: (i, 0)),
    )(numer2, den2, b2, Wl1, bl1, Wl2, bl2)

    return out[:n]


# speedup vs baseline: 6.3433x; 1.0596x over previous
import functools

import jax
import jax.numpy as jnp
from jax import lax
from jax.experimental import pallas as pl
from jax.experimental.pallas import tpu as pltpu
from jax.experimental.pallas import tpu_sc as plsc

BN = 256
NPAD = 53248
EPAD = 851968
W_WIN = 128
W_SLAB = 16
NB = 13
CELL_W = 37
PK_SENT = 1 << 28
PROWS = NB * 16 * CELL_W


def _prep1_body(x_ref, w_ref, asv_ref, adv_ref, h_ref, asT_ref, adT_ref):
    hb = jnp.dot(x_ref[...], w_ref[...], preferred_element_type=jnp.float32)
    h_ref[...] = hb
    h3 = hb.reshape(BN, 8, 64)
    asb = (h3 * asv_ref[...][None]).sum(-1)
    adb = (h3 * adv_ref[...][None]).sum(-1)
    asT_ref[...] = asb.T
    adT_ref[...] = adb.T


def _mid_body(num_ref, den_ref, b1_ref, w2_ref, as2v_ref, ad2v_ref,
              h2_ref, as2_ref, ad2_ref):
    den = den_ref[...].sum(-1)
    inv = 1.0 / (den + 1e-16)
    pieces = []
    for s in range(32):
        pieces.append(num_ref[s] * inv[s // 4][:, None])
    h1g = jnp.concatenate(pieces, axis=-1) + b1_ref[...][None]
    h1g = jnp.where(h1g > 0, h1g, jnp.exp(jnp.minimum(h1g, 0.0)) - 1.0)
    h2 = jnp.dot(h1g, w2_ref[...], preferred_element_type=jnp.float32)
    h2_ref[...] = h2
    as2_ref[...] = (h2 * as2v_ref[...]).sum(-1)
    ad2_ref[...] = (h2 * ad2v_ref[...]).sum(-1)


def _fin_body(num_ref, den_ref, b2_ref, wl1_ref, bl1_ref, wl2_ref, bl2_ref,
              o_ref):
    num = jnp.concatenate([num_ref[0], num_ref[1]], axis=-1)
    den = den_ref[0].sum(-1)
    h = num * (1.0 / (den + 1e-16))[:, None] + b2_ref[...][None]
    z = jnp.maximum(
        jnp.dot(h, wl1_ref[...], preferred_element_type=jnp.float32)
        + bl1_ref[...][None], 0.0)
    o_ref[...] = (jnp.dot(z, wl2_ref[...], preferred_element_type=jnp.float32)
                  + bl2_ref[...][None])


def _part_body(src2, dst2, ep, m1, sbuf, dbuf, cellb, sta, cnt_s, sem):
    t = lax.axis_index("s")
    sent = jnp.full((16,), PK_SENT, jnp.int32)

    def _prefill():
        @pl.loop(0, CELL_W)
        def _(r):
            for b in range(NB):
                for cc in range(0, W_WIN, 16):
                    sta[b, r, pl.ds(cc, 16)] = sent

    _prefill()
    for b in range(NB):
        cnt_s[b] = 0

    @pl.loop(0, 52)
    def _sb(sb):
        rowbase = t * 416 + sb * 8
        pltpu.sync_copy(src2.at[pl.ds(rowbase, 8)], sbuf)
        pltpu.sync_copy(dst2.at[pl.ds(rowbase, 8)], dbuf)

        @pl.loop(0, 8)
        def _rw(r):
          for cc in range(0, W_WIN, 16):
            sv = sbuf[r, pl.ds(cc, 16)]
            dv = dbuf[r, pl.ds(cc, 16)]
            pk = sv | ((dv & 4095) << 16)
            bv = dv >> 12
            for b in range(NB):
                m = bv == b
                pos = plsc.cumsum(jnp.where(m, 1, 0))
                cnt = cnt_s[b]
                idx = cnt + pos - 1
                plsc.store_scatter(sta, [jnp.full((16,), b, jnp.int32),
                                         idx >> 7, idx & 127], pk, mask=m)
                cnt_s[b] = cnt + pos[15]

    for b in range(NB):
        pltpu.sync_copy(sta.at[b], m1.at[pl.ds((b * 16 + t) * CELL_W, CELL_W)])

    plsc.subcore_barrier()

    _prefill()
    for b in range(NB):
        cnt_s[b] = 0

    for b in range(NB):
        @pl.loop(0, 16)
        def _cell(u):
            pltpu.sync_copy(m1.at[pl.ds((b * 16 + u) * CELL_W, CELL_W)],
                            cellb)

            @pl.loop(0, CELL_W)
            def _row(r):
                for cc in range(0, W_WIN, 16):
                    pk = cellb[r, pl.ds(cc, 16)]
                    m = ((pk >> 28) == 0) & (((pk >> 24) & 15) == t)
                    pos = plsc.cumsum(jnp.where(m, 1, 0))
                    cnt = cnt_s[b]
                    idx = cnt + pos - 1
                    plsc.store_scatter(sta, [jnp.full((16,), b, jnp.int32),
                                             idx >> 7, idx & 127], pk,
                                       mask=m)
                    cnt_s[b] = cnt + pos[15]

    for b in range(NB):
        pltpu.sync_copy(sta.at[b], ep.at[pl.ds((b * 16 + t) * CELL_W, CELL_W)])


def _make_gat_body(mult, nslab):
    def body(ep, asT3, adT3, hv, z2d,
             numer, den,
             pbuf, gbuf, rowbuf, rowb2, wbuf, as_t, ad_t, acc, densw,
             sema, semb):
        c = lax.axis_index("c")
        t = lax.axis_index("s")
        iota = lax.iota(jnp.int32, 16)

        @pl.loop(0, nslab)
        def _slab(sl):
            head = sl // 4
            pltpu.sync_copy(asT3.at[head], as_t)
            pltpu.sync_copy(adT3.at[head], ad_t)

            def _bucket(bi, carry):
                b = bi * 2 + c
                base = b * 4096
                pltpu.sync_copy(z2d, acc)

                @pl.when(sl % 4 == 0)
                def _():
                    pltpu.sync_copy(z2d, densw)

                pltpu.sync_copy(ep.at[pl.ds((b * 16 + t) * CELL_W, CELL_W)],
                                pbuf)

                @pl.loop(0, CELL_W)
                def _gidx(r):
                    for cc in range(0, W_WIN, 16):
                        pk = pbuf[r, pl.ds(cc, 16)]
                        gbuf[r, pl.ds(cc, 16)] = (pk & 65535) * mult + sl

                def _proc(wi, rb):
                    for j in range(8):
                        pk = pbuf[wi, pl.ds(j * 16, 16)]
                        sv = pk & 65535
                        dv = base + ((pk >> 16) & 4095)
                        a = plsc.load_gather(as_t, [sv >> 7, sv & 127])
                        bq = plsc.load_gather(ad_t, [dv >> 7, dv & 127])
                        tt = a + bq
                        tt = jnp.where(tt >= 0, tt, tt * jnp.float32(0.2))
                        wm = jnp.where((pk >> 28) == 0, jnp.float32(1.0),
                                       jnp.float32(0.0))
                        wbuf[pl.ds(j * 16, 16)] = jnp.exp(tt) * wm
                    pltpu.make_async_copy(hv.at[gbuf.at[wi]], rb,
                                          sema if rb is rowbuf else semb
                                          ).wait()
                    for g in range(8):
                        wv = wbuf[pl.ds(g * 16, 16)]
                        rv = (pbuf[wi, pl.ds(g * 16, 16)] >> 16) & 255

                        @pl.when(sl % 4 == 0)
                        def _():
                            plsc.addupdate_scatter(densw, [rv, iota], wv)

                        for j in range(16):
                            r = g * 16 + j
                            wsc = wv[j]
                            rows = jnp.full((16,), rv[j], jnp.int32)
                            v = rb[r, pl.ds(0, 16)] * wsc
                            plsc.addupdate_scatter(acc, [rows, iota], v)

                pltpu.async_copy(hv.at[gbuf.at[0]], rowbuf, sema)

                @pl.loop(0, (CELL_W - 1) // 2)
                def _win(k):
                    wi = 2 * k
                    pltpu.async_copy(hv.at[gbuf.at[wi + 1]], rowb2, semb)
                    _proc(wi, rowbuf)
                    pltpu.async_copy(hv.at[gbuf.at[wi + 2]], rowbuf, sema)
                    _proc(wi + 1, rowb2)

                _proc(CELL_W - 1, rowbuf)

                pltpu.sync_copy(acc,
                                numer.at[sl, pl.ds(base + t * 256, 256)])

                @pl.when(sl % 4 == 0)
                def _():
                    pltpu.sync_copy(densw,
                                    den.at[head, pl.ds(base + t * 256, 256)])

                return carry

            lax.fori_loop(0, 7 - c, _bucket, 0)

    return body


_SC_MESH = plsc.VectorSubcoreMesh(core_axis_name="c", subcore_axis_name="s")
_SC_MESH1 = plsc.VectorSubcoreMesh(core_axis_name="c", subcore_axis_name="s",
                                   num_cores=1)

_SC_PARAMS = pltpu.CompilerParams(needs_layout_passes=False,
                                  use_tc_tiling_on_sc=False)

_partition = functools.partial(
    pl.kernel, _part_body, mesh=_SC_MESH1, compiler_params=_SC_PARAMS,
    name="gat_partition",
    out_type=(jax.ShapeDtypeStruct((PROWS, W_WIN), jnp.int32),
              jax.ShapeDtypeStruct((PROWS, W_WIN), jnp.int32)),
    scratch_types=[
        pltpu.VMEM((8, W_WIN), jnp.int32),
        pltpu.VMEM((8, W_WIN), jnp.int32),
        pltpu.VMEM((CELL_W, W_WIN), jnp.int32),
        pltpu.VMEM((NB, CELL_W, W_WIN), jnp.int32),
        pltpu.SMEM((16,), jnp.int32),
        pltpu.SemaphoreType.DMA,
    ])


def _gat_scratch():
    return [
        pltpu.VMEM((CELL_W, W_WIN), jnp.int32),
        pltpu.VMEM((CELL_W, W_WIN), jnp.int32),
        pltpu.VMEM((W_WIN, W_SLAB), jnp.float32),
        pltpu.VMEM((W_WIN, W_SLAB), jnp.float32),
        pltpu.VMEM((W_WIN,), jnp.float32),
        pltpu.VMEM((392, 128), jnp.float32),
        pltpu.VMEM((392, 128), jnp.float32),
        pltpu.VMEM((256, W_SLAB), jnp.float32),
        pltpu.VMEM((256, W_SLAB), jnp.float32),
        pltpu.SemaphoreType.DMA,
        pltpu.SemaphoreType.DMA,
    ]


_gat1 = functools.partial(
    pl.kernel, _make_gat_body(32, 32), mesh=_SC_MESH,
    compiler_params=_SC_PARAMS, name="gat_edges_l1",
    out_type=(jax.ShapeDtypeStruct((32, NPAD, W_SLAB), jnp.float32),
              jax.ShapeDtypeStruct((8, NPAD, W_SLAB), jnp.float32)),
    scratch_types=_gat_scratch())

_gat2 = functools.partial(
    pl.kernel, _make_gat_body(2, 2), mesh=_SC_MESH,
    compiler_params=_SC_PARAMS, name="gat_edges_l2",
    out_type=(jax.ShapeDtypeStruct((2, NPAD, W_SLAB), jnp.float32),
              jax.ShapeDtypeStruct((1, NPAD, W_SLAB), jnp.float32)),
    scratch_types=_gat_scratch())


def kernel(x, edge_index, W1, a_src1, a_dst1, b1, W2, a_src2, a_dst2, b2,
           Wl1, bl1, Wl2, bl2):
    n = x.shape[0]
    e = edge_index.shape[1]
    grid = NPAD // BN

    loop = jnp.arange(n, dtype=jnp.int32)
    npad_e = EPAD - (e + n)
    pad_ar = jnp.arange(npad_e, dtype=jnp.int32) % 128
    src_full = jnp.concatenate([edge_index[0].astype(jnp.int32), loop, pad_ar])
    dst_full = jnp.concatenate([edge_index[1].astype(jnp.int32), loop,
                                n + pad_ar])
    src2 = src_full.reshape(EPAD // W_WIN, W_WIN)
    dst2 = dst_full.reshape(EPAD // W_WIN, W_WIN)

    xp = jnp.pad(x, ((0, NPAD - n), (0, 0)))

    ep, _m1 = _partition()(src2, dst2)

    h1, asT, adT = pl.pallas_call(
        _prep1_body,
        out_shape=(jax.ShapeDtypeStruct((NPAD, 512), jnp.float32),
                   jax.ShapeDtypeStruct((8, NPAD), jnp.float32),
                   jax.ShapeDtypeStruct((8, NPAD), jnp.float32)),
        grid=(grid,),
        in_specs=[
            pl.BlockSpec((BN, 75), lambda i: (i, 0)),
            pl.BlockSpec((75, 512), lambda i: (0, 0)),
            pl.BlockSpec((8, 64), lambda i: (0, 0)),
            pl.BlockSpec((8, 64), lambda i: (0, 0)),
        ],
        out_specs=(
            pl.BlockSpec((BN, 512), lambda i: (i, 0)),
            pl.BlockSpec((8, BN), lambda i: (0, i)),
            pl.BlockSpec((8, BN), lambda i: (0, i)),
        ),
    )(xp, W1, a_src1, a_dst1)

    h1v = h1.reshape(NPAD * 32, W_SLAB)
    z2d = jnp.zeros((256, W_SLAB), jnp.float32)
    asT3 = asT[:, :392 * 128].reshape(8, 392, 128)
    adT3 = adT[:, :392 * 128].reshape(8, 392, 128)
    numer1, den1 = _gat1()(ep, asT3, adT3, h1v, z2d)

    h2, as2, ad2 = pl.pallas_call(
        _mid_body,
        out_shape=(jax.ShapeDtypeStruct((NPAD, 32), jnp.float32),
                   jax.ShapeDtypeStruct((NPAD,), jnp.float32),
                   jax.ShapeDtypeStruct((NPAD,), jnp.float32)),
        grid=(grid,),
        in_specs=[
            pl.BlockSpec((32, BN, W_SLAB), lambda i: (0, i, 0)),
            pl.BlockSpec((8, BN, W_SLAB), lambda i: (0, i, 0)),
            pl.BlockSpec((512,), lambda i: (0,)),
            pl.BlockSpec((512, 32), lambda i: (0, 0)),
            pl.BlockSpec((1, 32), lambda i: (0, 0)),
            pl.BlockSpec((1, 32), lambda i: (0, 0)),
        ],
        out_specs=(
            pl.BlockSpec((BN, 32), lambda i: (i, 0)),
            pl.BlockSpec((BN,), lambda i: (i,)),
            pl.BlockSpec((BN,), lambda i: (i,)),
        ),
    )(numer1, den1, b1, W2, a_src2, a_dst2)

    h2v = h2.reshape(NPAD * 2, W_SLAB)
    asT3_2 = as2[:392 * 128].reshape(1, 392, 128)
    adT3_2 = ad2[:392 * 128].reshape(1, 392, 128)
    numer2, den2 = _gat2()(ep, asT3_2, adT3_2, h2v, z2d)

    out = pl.pallas_call(
        _fin_body,
        out_shape=jax.ShapeDtypeStruct((NPAD, 1), jnp.float32),
        grid=(grid,),
        in_specs=[
            pl.BlockSpec((2, BN, W_SLAB), lambda i: (0, i, 0)),
            pl.BlockSpec((1, BN, W_SLAB), lambda i: (0, i, 0)),
            pl.BlockSpec((32,), lambda i: (0,)),
            pl.BlockSpec((32, 16), lambda i: (0, 0)),
            pl.BlockSpec((16,), lambda i: (0,)),
            pl.BlockSpec((16, 1), lambda i: (0, 0)),
            pl.BlockSpec((1,), lambda i: (0,)),
        ],
        out_specs=pl.BlockSpec((BN, 1), lambda i: (i, 0)),
    )(numer2, den2, b2, Wl1, bl1, Wl2, bl2)

    return out[:n]
